# baseline (speedup 1.0000x reference)
import functools
import math

import jax
import jax.numpy as jnp
from jax import lax
from jax.experimental import pallas as pl
from jax.experimental.pallas import tpu as pltpu
from jax.experimental.pallas import tpu_sc as plsc

_INFO = plsc.get_sparse_core_info()
_NC = _INFO.num_cores
_NS = _INFO.num_subcores
_NW = _NC * _NS
_L = _INFO.num_lanes

_G = 128


@functools.partial(jax.jit, static_argnames=("n_units",))
def _run(idx2d, table, n_units):
    d = table.shape[1]
    nrows = idx2d.shape[0] * d

    @functools.partial(
        pl.kernel,
        out_type=jax.ShapeDtypeStruct((nrows, _G), jnp.float32),
        mesh=plsc.VectorSubcoreMesh(core_axis_name="c", subcore_axis_name="s"),
        scratch_types=[
            pltpu.VMEM((n_units, _G), jnp.int32),
            [pltpu.VMEM((_G, d), jnp.float32) for _ in range(2)],
            [pltpu.VMEM((d, _G), jnp.float32) for _ in range(2)],
            [pltpu.SemaphoreType.DMA for _ in range(2)],
            [pltpu.SemaphoreType.DMA for _ in range(2)],
        ],
        compiler_params=pltpu.CompilerParams(
            use_tc_tiling_on_sc=False, needs_layout_passes=False
        ),
    )
    def emb(idx_hbm, table_hbm, out_hbm, idx_v, rowb, tbuf, gsems, ssems):
        wid = lax.axis_index("s") * _NC + lax.axis_index("c")
        scale = jnp.float32(math.sqrt(d))
        i16 = lax.iota(jnp.int32, 16)
        row_idx = [(i16 + 16 * k) for k in range(8)]

        pltpu.sync_copy(idx_hbm.at[pl.ds(wid * n_units, n_units)], idx_v)

        def fire(u, bi):
            pltpu.async_copy(
                table_hbm.at[idx_v.at[u]], rowb[bi], gsems[bi]
            )

        def drain(bi):
            pltpu.make_async_copy(
                table_hbm.at[pl.ds(0, _G)], rowb[bi], gsems[bi]
            ).wait()

        def drain_store(bi):
            pltpu.make_async_copy(
                tbuf[bi], out_hbm.at[pl.ds(0, d)], ssems[bi]
            ).wait()

        fire(0, 0)

        @pl.loop(0, n_units, step=2)
        def _unit(u0):
            for sub in range(2):
                u = u0 + sub
                g = wid * n_units + u

                @pl.when(u + 1 < n_units)
                def _pre():
                    fire(u + 1, 1 - sub)

                drain(sub)

                @pl.when(u >= 2)
                def _free():
                    drain_store(sub)

                @plsc.parallel_loop(0, d, unroll=8)
                def _tr(f):
                    col = jnp.broadcast_to(f.astype(jnp.int32), (16,))
                    for k in range(8):
                        vals = plsc.load_gather(rowb[sub], [row_idx[k], col])
                        tbuf[sub][f, pl.ds(16 * k, 16)] = vals * scale

                h = lax.shift_right_logical(g, 7)
                bb = lax.bitwise_and(g, _G - 1)
                for ft in range(d // 8):
                    pltpu.async_copy(
                        tbuf[sub].at[pl.ds(8 * ft, 8), pl.ds(0, _G)],
                        out_hbm.at[pl.ds(((h * (d // 8) + ft) * _G + bb) * 8, 8)],
                        ssems[sub],
                    )

        for bi in range(2):
            drain_store(bi)

    return emb(idx2d, table)


def kernel(x, table):
    batch, hist = x.shape
    d = table.shape[1]
    assert batch % _G == 0 and d % _L == 0 and d % 8 == 0
    n_blocks = hist * (batch // _G)
    assert n_blocks % (2 * _NW) == 0
    idx2d = x.astype(jnp.int32).T.reshape(n_blocks, _G)
    out2 = _run(idx2d, table, n_blocks // _NW)
    out = (
        out2.reshape(hist, d // 8, batch // _G, 8, _G)
        .transpose(2, 4, 0, 1, 3)
        .reshape(batch, hist, d)
    )
    return out


# speedup vs baseline: 1.0173x; 1.0173x over previous
import functools
import math

import jax
import jax.numpy as jnp
from jax import lax
from jax.experimental import pallas as pl
from jax.experimental.pallas import tpu as pltpu
from jax.experimental.pallas import tpu_sc as plsc

_INFO = plsc.get_sparse_core_info()
_NC = _INFO.num_cores
_NS = _INFO.num_subcores
_NW = _NC * _NS
_L = _INFO.num_lanes

_G = 128


@functools.partial(jax.jit, static_argnames=("n_units",))
def _run(idx2d, table, n_units):
    d = table.shape[1]
    nrows = idx2d.shape[0] * d

    @functools.partial(
        pl.kernel,
        out_type=jax.ShapeDtypeStruct((nrows, _G), jnp.float32),
        mesh=plsc.VectorSubcoreMesh(core_axis_name="c", subcore_axis_name="s"),
        scratch_types=[
            pltpu.VMEM((2 * n_units, _G), jnp.int32),
            [pltpu.VMEM((2 * _G, d), jnp.float32) for _ in range(2)],
            [pltpu.VMEM((2 * d, _G), jnp.float32) for _ in range(2)],
            [pltpu.SemaphoreType.DMA for _ in range(2)],
            [pltpu.SemaphoreType.DMA for _ in range(2)],
        ],
        compiler_params=pltpu.CompilerParams(
            use_tc_tiling_on_sc=False, needs_layout_passes=False
        ),
    )
    def emb(idx_hbm, table_hbm, out_hbm, idx_v, rowb, tbuf, gsems, ssems):
        wid = lax.axis_index("s") * _NC + lax.axis_index("c")
        scale = jnp.float32(math.sqrt(d))
        i16 = lax.iota(jnp.int32, 16)
        row_idx = [(i16 + 16 * k) for k in range(8)]

        pltpu.sync_copy(
            idx_hbm.at[pl.ds(wid * 2 * n_units, 2 * n_units)], idx_v
        )

        def fire(u, bi):
            for j in range(2):
                pltpu.async_copy(
                    table_hbm.at[idx_v.at[2 * u + j]],
                    rowb[bi].at[pl.ds(j * _G, _G)],
                    gsems[bi],
                )

        def drain(bi):
            pltpu.make_async_copy(
                table_hbm.at[pl.ds(0, 2 * _G)], rowb[bi], gsems[bi]
            ).wait()

        def drain_store(bi):
            pltpu.make_async_copy(
                tbuf[bi], out_hbm.at[pl.ds(0, 2 * d)], ssems[bi]
            ).wait()

        fire(0, 0)

        @pl.loop(0, n_units, step=2)
        def _unit(u0):
            for sub in range(2):
                u = u0 + sub
                g2 = wid * n_units + u
                h = lax.shift_right_logical(g2, 6)
                bbp = lax.bitwise_and(g2, 63)

                @pl.when(u + 1 < n_units)
                def _pre():
                    fire(u + 1, 1 - sub)

                drain(sub)

                @pl.when(u >= 2)
                def _free():
                    drain_store(sub)

                @plsc.parallel_loop(0, 2 * d, unroll=8)
                def _tr(r):
                    ri = r.astype(jnp.int32)
                    f = lax.shift_right_logical(ri, 4) * 8 + lax.bitwise_and(ri, 7)
                    boff = lax.shift_left(lax.bitwise_and(ri, 8), 4)
                    col = jnp.broadcast_to(f, (16,))
                    bvec = jnp.broadcast_to(boff, (16,))
                    for k in range(8):
                        vals = plsc.load_gather(
                            rowb[sub], [row_idx[k] + bvec, col]
                        )
                        tbuf[sub][ri, pl.ds(16 * k, 16)] = vals * scale

                for ft in range(d // 8):
                    pltpu.async_copy(
                        tbuf[sub].at[pl.ds(16 * ft, 16), pl.ds(0, _G)],
                        out_hbm.at[
                            pl.ds(((h * (d // 8) + ft) * _G + 2 * bbp) * 8, 16)
                        ],
                        ssems[sub],
                    )

        for bi in range(2):
            drain_store(bi)

    return emb(idx2d, table)


def kernel(x, table):
    batch, hist = x.shape
    d = table.shape[1]
    assert batch % _G == 0 and d % _L == 0 and d % 8 == 0
    n_blocks = hist * (batch // _G)
    assert n_blocks % (4 * _NW) == 0 and batch // _G == 128
    idx2d = x.astype(jnp.int32).T.reshape(n_blocks, _G)
    out2 = _run(idx2d, table, n_blocks // (2 * _NW))
    out = (
        out2.reshape(hist, d // 8, batch // _G, 8, _G)
        .transpose(2, 4, 0, 1, 3)
        .reshape(batch, hist, d)
    )
    return out


# speedup vs baseline: 1.0403x; 1.0227x over previous
import functools
import math

import jax
import jax.numpy as jnp
from jax import lax
from jax.experimental import pallas as pl
from jax.experimental.pallas import tpu as pltpu
from jax.experimental.pallas import tpu_sc as plsc

_INFO = plsc.get_sparse_core_info()
_NC = _INFO.num_cores
_NS = _INFO.num_subcores
_NW = _NC * _NS
_L = _INFO.num_lanes

_G = 128
_GPC = 2
_CHUNK = _G * _GPC
_NBUF = 4


@functools.partial(jax.jit, static_argnames=("n_chunks",))
def _run(idx2d, table, n_chunks):
    d = table.shape[1]
    b = idx2d.shape[0] * _G
    irows_pw = n_chunks * _GPC

    @functools.partial(
        pl.kernel,
        out_type=jax.ShapeDtypeStruct((b, d), jnp.float32),
        mesh=plsc.VectorSubcoreMesh(core_axis_name="c", subcore_axis_name="s"),
        scratch_types=[
            pltpu.VMEM((irows_pw, _G), jnp.int32),
            [pltpu.VMEM((_CHUNK, d), jnp.float32) for _ in range(_NBUF)],
            [pltpu.SemaphoreType.DMA for _ in range(_NBUF)],
            [pltpu.SemaphoreType.DMA for _ in range(_NBUF)],
        ],
        compiler_params=pltpu.CompilerParams(use_tc_tiling_on_sc=False),
    )
    def emb(idx_hbm, table_hbm, out_hbm, idx_v, rows, gsems, ssems):
        wid = lax.axis_index("s") * _NC + lax.axis_index("c")
        scale = jnp.float32(math.sqrt(d))
        pltpu.sync_copy(idx_hbm.at[pl.ds(wid * irows_pw, irows_pw)], idx_v)

        def fire_gathers(cc, bi):
            for j in range(_GPC):
                pltpu.async_copy(
                    table_hbm.at[idx_v.at[cc * _GPC + j]],
                    rows[bi].at[pl.ds(j * _G, _G)],
                    gsems[bi],
                )

        def drain_gathers(bi):
            pltpu.make_async_copy(
                table_hbm.at[pl.ds(0, _CHUNK)], rows[bi], gsems[bi]
            ).wait()

        def drain_scatter(bi):
            pltpu.make_async_copy(
                rows[bi], out_hbm.at[pl.ds(0, _CHUNK)], ssems[bi]
            ).wait()

        fire_gathers(0, 0)
        fire_gathers(1, 1)

        @pl.loop(0, n_chunks, step=_NBUF)
        def _step(c):
            for bi in range(_NBUF):
                cc = c + bi
                drain_gathers(bi)

                nbi = (bi + 2) % _NBUF

                @pl.when(cc + 2 < n_chunks)
                def _prefetch():
                    @pl.when(cc >= 2)
                    def _free():
                        drain_scatter(nbi)

                    fire_gathers(cc + 2, nbi)

                @plsc.parallel_loop(0, _CHUNK, unroll=8)
                def _scale(r):
                    for q in range(d // _L):
                        sl = pl.ds(q * _L, _L)
                        rows[bi][r, sl] = rows[bi][r, sl] * scale

                pltpu.async_copy(
                    rows[bi],
                    out_hbm.at[pl.ds((wid * n_chunks + cc) * _CHUNK, _CHUNK)],
                    ssems[bi],
                )

        for bi in range(_NBUF):
            drain_scatter(bi)

    return emb(idx2d, table)


def kernel(x, table):
    batch, hist = x.shape
    d = table.shape[1]
    b = batch * hist
    assert b % (_NW * _CHUNK * _NBUF) == 0 and d % _L == 0
    idx2d = x.astype(jnp.int32).reshape(b // _G, _G)
    n_chunks = b // (_NW * _CHUNK)
    out = _run(idx2d, table, n_chunks)
    return out.reshape(batch, hist, d)
